# baseline (speedup 1.0000x reference)
import functools

import jax
import jax.numpy as jnp
from jax import lax
from jax.experimental import pallas as pl
from jax.experimental.pallas import tpu as pltpu
from jax.experimental.pallas import tpu_sc as plsc

D = 128
H = 64
C = 64
M = 32
SC0_W = 3
SC1_W = 1
NW = 32
LANES = 16
KMAX = 4

HALVES = {"bond": (0, 1), "angle": (0, 0, 1), "torsion": (0, 0, 1, 1)}


def _proj_body(x_ref, w_ref, t_ref):
    t_ref[...] = jnp.dot(x_ref[...], w_ref[0],
                         preferred_element_type=jnp.float32)


def _project(x, wstack):
    n_atoms = x.shape[0]
    blk = 5000
    nblk = n_atoms // blk
    npiece = wstack.shape[0]
    return pl.pallas_call(
        _proj_body,
        grid=(nblk, npiece),
        in_specs=[
            pl.BlockSpec((blk, D), lambda i, h: (i, 0)),
            pl.BlockSpec((1, D, D), lambda i, h: (h, 0, 0)),
        ],
        out_specs=pl.BlockSpec((blk, D), lambda i, h: (h * nblk + i, 0)),
        out_shape=jax.ShapeDtypeStruct((npiece * n_atoms, D), jnp.float32),
    )(x, wstack)


def _sc_gather_body(table, bidx, aidx, tidx, zb, za, zt,
                    ibig, gbuf0, gbuf1, zbuf0, zbuf1,
                    gsem0, gsem1, ssem0, ssem1):
    cid = lax.axis_index("c")
    sid = lax.axis_index("s")
    prefix = (1 - cid) * (SC0_W * sid) + cid * (16 * SC0_W + SC1_W * sid)
    weight = SC0_W - (SC0_W - SC1_W) * cid
    units = 16 * (SC0_W + SC1_W)

    def drain(sem, n):
        for _ in range(n):
            pltpu.make_async_copy(table.at[pl.ds(0, C)], zbuf0, sem).wait()

    def do_type(idx_hbm, z_hbm, halves):
        k = len(halves)
        n_chunks = z_hbm.shape[0]
        qmax = (n_chunks * SC0_W) // (16 * (SC0_W + SC1_W))
        npad_row = (n_chunks + qmax) * C
        qc = qmax * C
        start = (n_chunks * prefix) // units
        qh = ((n_chunks * weight) // units) // 2
        for p in range(k):
            pltpu.sync_copy(idx_hbm.at[pl.ds(p * npad_row + start * C, qc)],
                            ibig.at[pl.ds(p * qc, qc)])

        def issue(g_rel, gbuf, gsem):
            for p in range(k):
                pltpu.async_copy(
                    table.at[ibig.at[pl.ds(p * qc + g_rel * C, C)]],
                    gbuf.at[pl.ds(p * C, C)], gsem)

        def accumulate(gbuf, zbuf):
            def acc_row(r, carry2):
                for j in range(H // LANES):
                    f = None
                    rv = None
                    for p, hf in enumerate(halves):
                        gf = gbuf[p * C + r, pl.ds(hf * H + j * LANES, LANES)]
                        gr = gbuf[p * C + r,
                                  pl.ds((1 - hf) * H + j * LANES, LANES)]
                        f = gf if f is None else f + gf
                        rv = gr if rv is None else rv + gr
                    zbuf[r, pl.ds(j * LANES, LANES)] = f
                    zbuf[r, pl.ds(H + j * LANES, LANES)] = rv
                return carry2
            lax.fori_loop(0, C, acc_row, 0)

        issue(0, gbuf0, gsem0)
        issue(1, gbuf1, gsem1)

        def body(it, carry):
            g0 = 2 * it
            g1 = g0 + 1

            drain(gsem0, k)

            @pl.when(it > 0)
            def _():
                drain(ssem0, 1)

            accumulate(gbuf0, zbuf0)
            pltpu.async_copy(zbuf0, z_hbm.at[start + g0], ssem0)

            @pl.when(it + 1 < qh)
            def _():
                issue(g0 + 2, gbuf0, gsem0)

            drain(gsem1, k)

            @pl.when(it > 0)
            def _():
                drain(ssem1, 1)

            accumulate(gbuf1, zbuf1)
            pltpu.async_copy(zbuf1, z_hbm.at[start + g1], ssem1)

            @pl.when(it + 1 < qh)
            def _():
                issue(g1 + 2, gbuf1, gsem1)

            return carry

        lax.fori_loop(0, qh, body, 0)
        drain(ssem0, 1)
        drain(ssem1, 1)

    do_type(bidx, zb, HALVES["bond"])
    do_type(aidx, za, HALVES["angle"])
    do_type(tidx, zt, HALVES["torsion"])


def _sc_gather(table, bidx, aidx, tidx, ncb, nca, nct):
    mesh = plsc.VectorSubcoreMesh(core_axis_name="c", subcore_axis_name="s")
    out_type = [
        jax.ShapeDtypeStruct((ncb, C, 2 * H), jnp.float32),
        jax.ShapeDtypeStruct((nca, C, 2 * H), jnp.float32),
        jax.ShapeDtypeStruct((nct, C, 2 * H), jnp.float32),
    ]
    nidx_max = ((nct * SC0_W) // (16 * (SC0_W + SC1_W))) * C * KMAX
    scratch = [
        pltpu.VMEM((nidx_max,), jnp.int32),
        pltpu.VMEM((KMAX * C, 128), jnp.float32),
        pltpu.VMEM((KMAX * C, 128), jnp.float32),
        pltpu.VMEM((C, 2 * H), jnp.float32),
        pltpu.VMEM((C, 2 * H), jnp.float32),
        pltpu.SemaphoreType.DMA,
        pltpu.SemaphoreType.DMA,
        pltpu.SemaphoreType.DMA,
        pltpu.SemaphoreType.DMA,
    ]
    fn = pl.kernel(_sc_gather_body, out_type=out_type, mesh=mesh,
                   scratch_types=scratch,
                   compiler_params=pltpu.CompilerParams(
                       use_tc_tiling_on_sc=True))
    return fn(table, bidx, aidx, tidx)


def _tail_body(z_ref, r_ref, wr_ref, b1_ref, w2_ref, b2_ref, w3_ref, b3_ref,
               o_ref):
    z = z_ref[...].reshape(M * C, 2 * H)
    zf = z[:, 0:H]
    zr = z[:, H:2 * H]
    base = r_ref[...] * wr_ref[...] + b1_ref[...]
    h1f = jnp.maximum(zf + base, 0.0)
    h1r = jnp.maximum(zr + base, 0.0)
    w2 = w2_ref[...]
    h2f = jnp.maximum(
        jnp.dot(h1f, w2, preferred_element_type=jnp.float32) + b2_ref[...], 0.0)
    h2r = jnp.maximum(
        jnp.dot(h1r, w2, preferred_element_type=jnp.float32) + b2_ref[...], 0.0)
    o_ref[...] = (jnp.dot(h2f + h2r, w3_ref[...],
                          preferred_element_type=jnp.float32) + b3_ref[...])


def _tail(z4, repr_, wr, b1, w2, b2, w3, b3):
    n = repr_.shape[0]
    grid = (n + M * C - 1) // (M * C)
    n_out = w3.shape[1]
    return pl.pallas_call(
        _tail_body,
        grid=(grid,),
        in_specs=[
            pl.BlockSpec((M, C, 2 * H), lambda i: (i, 0, 0)),
            pl.BlockSpec((M * C, 1), lambda i: (i, 0)),
            pl.BlockSpec((1, H), lambda i: (0, 0)),
            pl.BlockSpec((1, H), lambda i: (0, 0)),
            pl.BlockSpec((H, H), lambda i: (0, 0)),
            pl.BlockSpec((1, H), lambda i: (0, 0)),
            pl.BlockSpec((H, n_out), lambda i: (0, 0)),
            pl.BlockSpec((1, n_out), lambda i: (0, 0)),
        ],
        out_specs=pl.BlockSpec((M * C, n_out), lambda i: (i, 0)),
        out_shape=jax.ShapeDtypeStruct((n, n_out), jnp.float32),
    )(z4, repr_, wr, b1, w2, b2, w3, b3)


def _prep_idx(idx, offsets, npad):
    n, k = idx.shape
    shifted = idx.astype(jnp.int32) + jnp.asarray(offsets, jnp.int32)[None, :]
    return jnp.pad(shifted, ((0, npad - n), (0, 0))).T.reshape(-1)


def kernel(x, bond_idx, angle_idx, torsion_idx, bond_repr, angle_repr,
           torsion_repr, bond_params, angle_params, torsion_params):
    wb1 = bond_params[0]
    wa1 = angle_params[0]
    wt1 = torsion_params[0]
    na_ = x.shape[0]

    wstack = jnp.stack([
        jnp.concatenate([wb1[0:D], wb1[D:2 * D]], axis=1),
        jnp.concatenate([wa1[0:D], wa1[2 * D:3 * D]], axis=1),
        jnp.concatenate([wa1[D:2 * D], wa1[D:2 * D]], axis=1),
        jnp.concatenate([wt1[0:D], wt1[3 * D:4 * D]], axis=1),
        jnp.concatenate([wt1[D:2 * D], wt1[2 * D:3 * D]], axis=1),
    ])

    table = _project(x, wstack)

    nb, naf, nt = bond_idx.shape[0], angle_idx.shape[0], torsion_idx.shape[0]
    align = 128
    step = C * align
    ncb = ((nb + step - 1) // step) * align
    nca = ((naf + step - 1) // step) * align
    nct = ((nt + step - 1) // step) * align

    def rows_padded(nc):
        return (nc + (nc * SC0_W) // (16 * (SC0_W + SC1_W))) * C

    bidx = _prep_idx(bond_idx, [0, 0], rows_padded(ncb))
    aidx = _prep_idx(angle_idx, [na_, 2 * na_, na_], rows_padded(nca))
    tidx = _prep_idx(torsion_idx, [3 * na_, 4 * na_, 4 * na_, 3 * na_],
                     rows_padded(nct))

    zb, za, zt = _sc_gather(table, bidx, aidx, tidx, ncb, nca, nct)

    def tail_for(z4, repr_, params):
        w1, b1, w2, b2, w3, b3 = params
        wr = w1[-1:, :]
        return _tail(z4, repr_, wr, b1.reshape(1, H), w2, b2.reshape(1, H),
                     w3, (2.0 * b3).reshape(1, -1))

    ob = tail_for(zb, bond_repr, bond_params)
    oa = tail_for(za, angle_repr, angle_params)
    ot = tail_for(zt, torsion_repr, torsion_params)

    return jnp.concatenate([ob, oa, ot], axis=0)


# speedup vs baseline: 1.0682x; 1.0682x over previous
import functools

import jax
import jax.numpy as jnp
from jax import lax
from jax.experimental import pallas as pl
from jax.experimental.pallas import tpu as pltpu
from jax.experimental.pallas import tpu_sc as plsc

D = 128
H = 64
C = 64
M = 32
SC0_W = 2
SC1_W = 1
NW = 32
LANES = 16
KMAX = 4

HALVES = {"bond": (0, 1), "angle": (0, 0, 1), "torsion": (0, 0, 1, 1)}


def _proj_body(x_ref, w_ref, t_ref):
    t_ref[...] = jnp.dot(x_ref[...], w_ref[0],
                         preferred_element_type=jnp.float32)


def _project(x, wstack):
    n_atoms = x.shape[0]
    blk = 5000
    nblk = n_atoms // blk
    npiece = wstack.shape[0]
    return pl.pallas_call(
        _proj_body,
        grid=(nblk, npiece),
        in_specs=[
            pl.BlockSpec((blk, D), lambda i, h: (i, 0)),
            pl.BlockSpec((1, D, D), lambda i, h: (h, 0, 0)),
        ],
        out_specs=pl.BlockSpec((blk, D), lambda i, h: (h * nblk + i, 0)),
        out_shape=jax.ShapeDtypeStruct((npiece * n_atoms, D), jnp.float32),
    )(x, wstack)


def _sc_gather_body(table, bidx, aidx, tidx, zb, za, zt,
                    ibig, gbuf0, gbuf1, zbuf0, zbuf1,
                    gsem0, gsem1, ssem0, ssem1):
    cid = lax.axis_index("c")
    sid = lax.axis_index("s")
    prefix = (1 - cid) * (SC0_W * sid) + cid * (16 * SC0_W + SC1_W * sid)
    weight = SC0_W - (SC0_W - SC1_W) * cid
    units = 16 * (SC0_W + SC1_W)

    def drain(sem, n):
        for _ in range(n):
            pltpu.make_async_copy(table.at[pl.ds(0, C)], zbuf0, sem).wait()

    def do_type(idx_hbm, z_hbm, halves):
        k = len(halves)
        n_chunks = z_hbm.shape[0]
        qmax = (n_chunks * SC0_W) // (16 * (SC0_W + SC1_W))
        npad_row = (n_chunks + qmax) * C
        qc = qmax * C
        start = (n_chunks * prefix) // units
        qh = ((n_chunks * weight) // units) // 2
        for p in range(k):
            pltpu.sync_copy(idx_hbm.at[pl.ds(p * npad_row + start * C, qc)],
                            ibig.at[pl.ds(p * qc, qc)])

        def issue(g_rel, gbuf, gsem):
            for p in range(k):
                pltpu.async_copy(
                    table.at[ibig.at[pl.ds(p * qc + g_rel * C, C)]],
                    gbuf.at[pl.ds(p * C, C)], gsem)

        def accumulate(gbuf, zbuf):
            def acc_row(r, carry2):
                for j in range(H // LANES):
                    f = None
                    rv = None
                    for p, hf in enumerate(halves):
                        gf = gbuf[p * C + r, pl.ds(hf * H + j * LANES, LANES)]
                        gr = gbuf[p * C + r,
                                  pl.ds((1 - hf) * H + j * LANES, LANES)]
                        f = gf if f is None else f + gf
                        rv = gr if rv is None else rv + gr
                    zbuf[r, pl.ds(j * LANES, LANES)] = f
                    zbuf[r, pl.ds(H + j * LANES, LANES)] = rv
                return carry2
            lax.fori_loop(0, C, acc_row, 0)

        issue(0, gbuf0, gsem0)
        issue(1, gbuf1, gsem1)

        def body(it, carry):
            g0 = 2 * it
            g1 = g0 + 1

            drain(gsem0, k)

            @pl.when(it > 0)
            def _():
                drain(ssem0, 1)

            accumulate(gbuf0, zbuf0)
            pltpu.async_copy(zbuf0, z_hbm.at[start + g0], ssem0)

            @pl.when(it + 1 < qh)
            def _():
                issue(g0 + 2, gbuf0, gsem0)

            drain(gsem1, k)

            @pl.when(it > 0)
            def _():
                drain(ssem1, 1)

            accumulate(gbuf1, zbuf1)
            pltpu.async_copy(zbuf1, z_hbm.at[start + g1], ssem1)

            @pl.when(it + 1 < qh)
            def _():
                issue(g1 + 2, gbuf1, gsem1)

            return carry

        lax.fori_loop(0, qh, body, 0)
        drain(ssem0, 1)
        drain(ssem1, 1)

    do_type(bidx, zb, HALVES["bond"])
    do_type(aidx, za, HALVES["angle"])
    do_type(tidx, zt, HALVES["torsion"])


def _sc_gather(table, bidx, aidx, tidx, ncb, nca, nct):
    mesh = plsc.VectorSubcoreMesh(core_axis_name="c", subcore_axis_name="s")
    out_type = [
        jax.ShapeDtypeStruct((ncb, C, 2 * H), jnp.float32),
        jax.ShapeDtypeStruct((nca, C, 2 * H), jnp.float32),
        jax.ShapeDtypeStruct((nct, C, 2 * H), jnp.float32),
    ]
    nidx_max = ((nct * SC0_W) // (16 * (SC0_W + SC1_W))) * C * KMAX
    scratch = [
        pltpu.VMEM((nidx_max,), jnp.int32),
        pltpu.VMEM((KMAX * C, 128), jnp.float32),
        pltpu.VMEM((KMAX * C, 128), jnp.float32),
        pltpu.VMEM((C, 2 * H), jnp.float32),
        pltpu.VMEM((C, 2 * H), jnp.float32),
        pltpu.SemaphoreType.DMA,
        pltpu.SemaphoreType.DMA,
        pltpu.SemaphoreType.DMA,
        pltpu.SemaphoreType.DMA,
    ]
    fn = pl.kernel(_sc_gather_body, out_type=out_type, mesh=mesh,
                   scratch_types=scratch,
                   compiler_params=pltpu.CompilerParams(
                       use_tc_tiling_on_sc=True))
    return fn(table, bidx, aidx, tidx)


def _tail_body(z_ref, r_ref, wr_ref, b1_ref, w2_ref, b2_ref, w3_ref, b3_ref,
               o_ref):
    z = z_ref[...].reshape(M * C, 2 * H)
    zf = z[:, 0:H]
    zr = z[:, H:2 * H]
    base = r_ref[...] * wr_ref[...] + b1_ref[...]
    h1f = jnp.maximum(zf + base, 0.0)
    h1r = jnp.maximum(zr + base, 0.0)
    w2 = w2_ref[...]
    h2f = jnp.maximum(
        jnp.dot(h1f, w2, preferred_element_type=jnp.float32) + b2_ref[...], 0.0)
    h2r = jnp.maximum(
        jnp.dot(h1r, w2, preferred_element_type=jnp.float32) + b2_ref[...], 0.0)
    o_ref[...] = (jnp.dot(h2f + h2r, w3_ref[...],
                          preferred_element_type=jnp.float32) + b3_ref[...])


def _tail(z4, repr_, wr, b1, w2, b2, w3, b3):
    n = repr_.shape[0]
    grid = (n + M * C - 1) // (M * C)
    n_out = w3.shape[1]
    return pl.pallas_call(
        _tail_body,
        grid=(grid,),
        in_specs=[
            pl.BlockSpec((M, C, 2 * H), lambda i: (i, 0, 0)),
            pl.BlockSpec((M * C, 1), lambda i: (i, 0)),
            pl.BlockSpec((1, H), lambda i: (0, 0)),
            pl.BlockSpec((1, H), lambda i: (0, 0)),
            pl.BlockSpec((H, H), lambda i: (0, 0)),
            pl.BlockSpec((1, H), lambda i: (0, 0)),
            pl.BlockSpec((H, n_out), lambda i: (0, 0)),
            pl.BlockSpec((1, n_out), lambda i: (0, 0)),
        ],
        out_specs=pl.BlockSpec((M * C, n_out), lambda i: (i, 0)),
        out_shape=jax.ShapeDtypeStruct((n, n_out), jnp.float32),
    )(z4, repr_, wr, b1, w2, b2, w3, b3)


def _prep_idx(idx, offsets, npad):
    n, k = idx.shape
    shifted = idx.astype(jnp.int32) + jnp.asarray(offsets, jnp.int32)[None, :]
    return jnp.pad(shifted, ((0, npad - n), (0, 0))).T.reshape(-1)


def kernel(x, bond_idx, angle_idx, torsion_idx, bond_repr, angle_repr,
           torsion_repr, bond_params, angle_params, torsion_params):
    wb1 = bond_params[0]
    wa1 = angle_params[0]
    wt1 = torsion_params[0]
    na_ = x.shape[0]

    wstack = jnp.stack([
        jnp.concatenate([wb1[0:D], wb1[D:2 * D]], axis=1),
        jnp.concatenate([wa1[0:D], wa1[2 * D:3 * D]], axis=1),
        jnp.concatenate([wa1[D:2 * D], wa1[D:2 * D]], axis=1),
        jnp.concatenate([wt1[0:D], wt1[3 * D:4 * D]], axis=1),
        jnp.concatenate([wt1[D:2 * D], wt1[2 * D:3 * D]], axis=1),
    ])

    table = _project(x, wstack)

    nb, naf, nt = bond_idx.shape[0], angle_idx.shape[0], torsion_idx.shape[0]
    align = 32 * (SC0_W + SC1_W)
    step = C * align
    ncb = ((nb + step - 1) // step) * align
    nca = ((naf + step - 1) // step) * align
    nct = ((nt + step - 1) // step) * align

    def rows_padded(nc):
        return (nc + (nc * SC0_W) // (16 * (SC0_W + SC1_W))) * C

    bidx = _prep_idx(bond_idx, [0, 0], rows_padded(ncb))
    aidx = _prep_idx(angle_idx, [na_, 2 * na_, na_], rows_padded(nca))
    tidx = _prep_idx(torsion_idx, [3 * na_, 4 * na_, 4 * na_, 3 * na_],
                     rows_padded(nct))

    zb, za, zt = _sc_gather(table, bidx, aidx, tidx, ncb, nca, nct)

    def tail_for(z4, repr_, params):
        w1, b1, w2, b2, w3, b3 = params
        wr = w1[-1:, :]
        return _tail(z4, repr_, wr, b1.reshape(1, H), w2, b2.reshape(1, H),
                     w3, (2.0 * b3).reshape(1, -1))

    ob = tail_for(zb, bond_repr, bond_params)
    oa = tail_for(za, angle_repr, angle_params)
    ot = tail_for(zt, torsion_repr, torsion_params)

    return jnp.concatenate([ob, oa, ot], axis=0)


# speedup vs baseline: 3.3505x; 3.1367x over previous
import functools

import jax
import jax.numpy as jnp
from jax import lax
from jax.experimental import pallas as pl
from jax.experimental.pallas import tpu as pltpu
from jax.experimental.pallas import tpu_sc as plsc

D = 128
H = 64
C = 64
M = 32
SC0_W = 1
SC1_W = 1
NW = 32
LANES = 16
KMAX = 4

HALVES = {"bond": (0, 1), "angle": (0, 0, 1), "torsion": (0, 0, 1, 1)}


def _proj_body(x_ref, w_ref, t_ref):
    t_ref[...] = jnp.dot(x_ref[...], w_ref[0],
                         preferred_element_type=jnp.float32)


def _project(x, wstack):
    n_atoms = x.shape[0]
    blk = 5000
    nblk = n_atoms // blk
    npiece = wstack.shape[0]
    return pl.pallas_call(
        _proj_body,
        grid=(nblk, npiece),
        in_specs=[
            pl.BlockSpec((blk, D), lambda i, h: (i, 0)),
            pl.BlockSpec((1, D, D), lambda i, h: (h, 0, 0)),
        ],
        out_specs=pl.BlockSpec((blk, D), lambda i, h: (h * nblk + i, 0)),
        out_shape=jax.ShapeDtypeStruct((npiece * n_atoms, D), jnp.float32),
    )(x, wstack)


def _sc_gather_body(table, bidx, aidx, tidx, zb, za, zt,
                    ibig, gbuf0, gbuf1, zbuf0, zbuf1,
                    gsem0, gsem1, ssem0, ssem1):
    cid = lax.axis_index("c")
    sid = lax.axis_index("s")
    prefix = (1 - cid) * (SC0_W * sid) + cid * (16 * SC0_W + SC1_W * sid)
    weight = SC0_W - (SC0_W - SC1_W) * cid
    units = 16 * (SC0_W + SC1_W)

    def drain(sem, n):
        for _ in range(n):
            pltpu.make_async_copy(table.at[pl.ds(0, C)], zbuf0, sem).wait()

    def do_type(idx_hbm, z_hbm, halves):
        k = len(halves)
        n_chunks = z_hbm.shape[0]
        qmax = (n_chunks * SC0_W) // (16 * (SC0_W + SC1_W))
        npad_row = (n_chunks + qmax) * C
        qc = qmax * C
        start = (n_chunks * prefix) // units
        qh = ((n_chunks * weight) // units) // 2
        for p in range(k):
            pltpu.sync_copy(idx_hbm.at[pl.ds(p * npad_row + start * C, qc)],
                            ibig.at[pl.ds(p * qc, qc)])

        def issue(g_rel, gbuf, gsem):
            for p in range(k):
                pltpu.async_copy(
                    table.at[ibig.at[pl.ds(p * qc + g_rel * C, C)]],
                    gbuf.at[pl.ds(p * C, C)], gsem)

        def accumulate(gbuf, zbuf):
            def acc_row(r, carry2):
                for j in range(H // LANES):
                    f = None
                    rv = None
                    for p, hf in enumerate(halves):
                        gf = gbuf[p * C + r, pl.ds(hf * H + j * LANES, LANES)]
                        gr = gbuf[p * C + r,
                                  pl.ds((1 - hf) * H + j * LANES, LANES)]
                        f = gf if f is None else f + gf
                        rv = gr if rv is None else rv + gr
                    zbuf[r, pl.ds(j * LANES, LANES)] = f
                    zbuf[r, pl.ds(H + j * LANES, LANES)] = rv
                return carry2
            lax.fori_loop(0, C, acc_row, 0)

        issue(0, gbuf0, gsem0)
        issue(1, gbuf1, gsem1)

        def body(it, carry):
            g0 = 2 * it
            g1 = g0 + 1

            drain(gsem0, k)

            @pl.when(it > 0)
            def _():
                drain(ssem0, 1)

            accumulate(gbuf0, zbuf0)
            pltpu.async_copy(zbuf0, z_hbm.at[start + g0], ssem0)

            @pl.when(it + 1 < qh)
            def _():
                issue(g0 + 2, gbuf0, gsem0)

            drain(gsem1, k)

            @pl.when(it > 0)
            def _():
                drain(ssem1, 1)

            accumulate(gbuf1, zbuf1)
            pltpu.async_copy(zbuf1, z_hbm.at[start + g1], ssem1)

            @pl.when(it + 1 < qh)
            def _():
                issue(g1 + 2, gbuf1, gsem1)

            return carry

        lax.fori_loop(0, qh, body, 0)
        drain(ssem0, 1)
        drain(ssem1, 1)

    do_type(bidx, zb, HALVES["bond"])
    do_type(aidx, za, HALVES["angle"])
    do_type(tidx, zt, HALVES["torsion"])


def _sc_gather(table, bidx, aidx, tidx, ncb, nca, nct):
    mesh = plsc.VectorSubcoreMesh(core_axis_name="c", subcore_axis_name="s")
    out_type = [
        jax.ShapeDtypeStruct((ncb, C, 2 * H), jnp.float32),
        jax.ShapeDtypeStruct((nca, C, 2 * H), jnp.float32),
        jax.ShapeDtypeStruct((nct, C, 2 * H), jnp.float32),
    ]
    nidx_max = ((nct * SC0_W) // (16 * (SC0_W + SC1_W))) * C * KMAX
    scratch = [
        pltpu.VMEM((nidx_max,), jnp.int32),
        pltpu.VMEM((KMAX * C, 128), jnp.float32),
        pltpu.VMEM((KMAX * C, 128), jnp.float32),
        pltpu.VMEM((C, 2 * H), jnp.float32),
        pltpu.VMEM((C, 2 * H), jnp.float32),
        pltpu.SemaphoreType.DMA,
        pltpu.SemaphoreType.DMA,
        pltpu.SemaphoreType.DMA,
        pltpu.SemaphoreType.DMA,
    ]
    fn = pl.kernel(_sc_gather_body, out_type=out_type, mesh=mesh,
                   scratch_types=scratch,
                   compiler_params=pltpu.CompilerParams(
                       use_tc_tiling_on_sc=True))
    return fn(table, bidx, aidx, tidx)


def _tail_body(z_ref, r_ref, wr_ref, b1_ref, w2_ref, b2_ref, w3_ref, b3_ref,
               o_ref):
    z = z_ref[...].reshape(M * C, 2 * H)
    zf = z[:, 0:H]
    zr = z[:, H:2 * H]
    base = r_ref[...] * wr_ref[...] + b1_ref[...]
    h1f = jnp.maximum(zf + base, 0.0)
    h1r = jnp.maximum(zr + base, 0.0)
    w2 = w2_ref[...]
    h2f = jnp.maximum(
        jnp.dot(h1f, w2, preferred_element_type=jnp.float32) + b2_ref[...], 0.0)
    h2r = jnp.maximum(
        jnp.dot(h1r, w2, preferred_element_type=jnp.float32) + b2_ref[...], 0.0)
    o_ref[...] = (jnp.dot(h2f + h2r, w3_ref[...],
                          preferred_element_type=jnp.float32) + b3_ref[...])


def _tail(z4, repr_, wr, b1, w2, b2, w3, b3):
    n = repr_.shape[0]
    grid = (n + M * C - 1) // (M * C)
    n_out = w3.shape[1]
    return pl.pallas_call(
        _tail_body,
        grid=(grid,),
        in_specs=[
            pl.BlockSpec((M, C, 2 * H), lambda i: (i, 0, 0)),
            pl.BlockSpec((M * C, 1), lambda i: (i, 0)),
            pl.BlockSpec((1, H), lambda i: (0, 0)),
            pl.BlockSpec((1, H), lambda i: (0, 0)),
            pl.BlockSpec((H, H), lambda i: (0, 0)),
            pl.BlockSpec((1, H), lambda i: (0, 0)),
            pl.BlockSpec((H, n_out), lambda i: (0, 0)),
            pl.BlockSpec((1, n_out), lambda i: (0, 0)),
        ],
        out_specs=pl.BlockSpec((M * C, n_out), lambda i: (i, 0)),
        out_shape=jax.ShapeDtypeStruct((n, n_out), jnp.float32),
    )(z4, repr_, wr, b1, w2, b2, w3, b3)


def _prep_idx(idx, offsets, npad, n_atoms):
    n, k = idx.shape
    off = jnp.asarray(offsets, jnp.int32)[None, :]
    shifted = idx.astype(jnp.int32) + off
    padv = (jnp.arange(npad - n, dtype=jnp.int32)[:, None] % n_atoms) + off
    return jnp.concatenate([shifted, padv], axis=0).T.reshape(-1)


def kernel(x, bond_idx, angle_idx, torsion_idx, bond_repr, angle_repr,
           torsion_repr, bond_params, angle_params, torsion_params):
    wb1 = bond_params[0]
    wa1 = angle_params[0]
    wt1 = torsion_params[0]
    na_ = x.shape[0]

    wstack = jnp.stack([
        jnp.concatenate([wb1[0:D], wb1[D:2 * D]], axis=1),
        jnp.concatenate([wa1[0:D], wa1[2 * D:3 * D]], axis=1),
        jnp.concatenate([wa1[D:2 * D], wa1[D:2 * D]], axis=1),
        jnp.concatenate([wt1[0:D], wt1[3 * D:4 * D]], axis=1),
        jnp.concatenate([wt1[D:2 * D], wt1[2 * D:3 * D]], axis=1),
    ])

    table = _project(x, wstack)

    nb, naf, nt = bond_idx.shape[0], angle_idx.shape[0], torsion_idx.shape[0]
    align = 32 * (SC0_W + SC1_W)
    step = C * align
    ncb = ((nb + step - 1) // step) * align
    nca = ((naf + step - 1) // step) * align
    nct = ((nt + step - 1) // step) * align

    def rows_padded(nc):
        return (nc + (nc * SC0_W) // (16 * (SC0_W + SC1_W))) * C

    bidx = _prep_idx(bond_idx, [0, 0], rows_padded(ncb), na_)
    aidx = _prep_idx(angle_idx, [na_, 2 * na_, na_], rows_padded(nca), na_)
    tidx = _prep_idx(torsion_idx, [3 * na_, 4 * na_, 4 * na_, 3 * na_],
                     rows_padded(nct), na_)

    zb, za, zt = _sc_gather(table, bidx, aidx, tidx, ncb, nca, nct)

    def tail_for(z4, repr_, params):
        w1, b1, w2, b2, w3, b3 = params
        wr = w1[-1:, :]
        return _tail(z4, repr_, wr, b1.reshape(1, H), w2, b2.reshape(1, H),
                     w3, (2.0 * b3).reshape(1, -1))

    ob = tail_for(zb, bond_repr, bond_params)
    oa = tail_for(za, angle_repr, angle_params)
    ot = tail_for(zt, torsion_repr, torsion_params)

    return jnp.concatenate([ob, oa, ot], axis=0)


# speedup vs baseline: 3.7261x; 1.1121x over previous
import functools

import jax
import jax.numpy as jnp
from jax import lax
from jax.experimental import pallas as pl
from jax.experimental.pallas import tpu as pltpu
from jax.experimental.pallas import tpu_sc as plsc

D = 128
H = 64
C = 64
M = 32
SC0_W = 1
SC1_W = 1
NW = 32
LANES = 16
KMAX = 4

HALVES = {"bond": (0, 1), "angle": (0, 0, 1), "torsion": (0, 0, 1, 1)}


def _proj_body(x_ref, w_ref, t_ref):
    t_ref[...] = jnp.dot(x_ref[...], w_ref[0],
                         preferred_element_type=jnp.float32)


def _project(x, wstack):
    n_atoms = x.shape[0]
    blk = 5000
    nblk = n_atoms // blk
    npiece = wstack.shape[0]
    return pl.pallas_call(
        _proj_body,
        grid=(nblk, npiece),
        in_specs=[
            pl.BlockSpec((blk, D), lambda i, h: (i, 0)),
            pl.BlockSpec((1, D, D), lambda i, h: (h, 0, 0)),
        ],
        out_specs=pl.BlockSpec((blk, D), lambda i, h: (h * nblk + i, 0)),
        out_shape=jax.ShapeDtypeStruct((npiece * n_atoms, D), jnp.float32),
    )(x, wstack)


def _sc_gather_body(halves, table, idx_hbm0, z_hbm0,
                    ibig, gbuf0, gbuf1, zbuf0, zbuf1,
                    gsem0, gsem1, ssem0, ssem1):
    cid = lax.axis_index("c")
    sid = lax.axis_index("s")
    prefix = (1 - cid) * (SC0_W * sid) + cid * (16 * SC0_W + SC1_W * sid)
    weight = SC0_W - (SC0_W - SC1_W) * cid
    units = 16 * (SC0_W + SC1_W)

    def drain(sem, n):
        for _ in range(n):
            pltpu.make_async_copy(table.at[pl.ds(0, C)], zbuf0, sem).wait()

    def do_type(idx_hbm, z_hbm, halves):
        k = len(halves)
        n_chunks = z_hbm.shape[0]
        qmax = (n_chunks * SC0_W) // (16 * (SC0_W + SC1_W))
        npad_row = (n_chunks + qmax) * C
        qc = qmax * C
        start = (n_chunks * prefix) // units
        qh = ((n_chunks * weight) // units) // 2
        for p in range(k):
            pltpu.sync_copy(idx_hbm.at[pl.ds(p * npad_row + start * C, qc)],
                            ibig.at[pl.ds(p * qc, qc)])

        def issue(g_rel, gbuf, gsem):
            for p in range(k):
                pltpu.async_copy(
                    table.at[ibig.at[pl.ds(p * qc + g_rel * C, C)]],
                    gbuf.at[pl.ds(p * C, C)], gsem)

        def accumulate(gbuf, zbuf):
            def acc_row(r, carry2):
                for j in range(H // LANES):
                    f = None
                    rv = None
                    for p, hf in enumerate(halves):
                        gf = gbuf[p * C + r, pl.ds(hf * H + j * LANES, LANES)]
                        gr = gbuf[p * C + r,
                                  pl.ds((1 - hf) * H + j * LANES, LANES)]
                        f = gf if f is None else f + gf
                        rv = gr if rv is None else rv + gr
                    zbuf[r, pl.ds(j * LANES, LANES)] = f
                    zbuf[r, pl.ds(H + j * LANES, LANES)] = rv
                return carry2
            lax.fori_loop(0, C, acc_row, 0)

        issue(0, gbuf0, gsem0)
        issue(1, gbuf1, gsem1)

        def body(it, carry):
            g0 = 2 * it
            g1 = g0 + 1

            drain(gsem0, k)

            @pl.when(it > 0)
            def _():
                drain(ssem0, 1)

            accumulate(gbuf0, zbuf0)
            pltpu.async_copy(zbuf0, z_hbm.at[start + g0], ssem0)

            @pl.when(it + 1 < qh)
            def _():
                issue(g0 + 2, gbuf0, gsem0)

            drain(gsem1, k)

            @pl.when(it > 0)
            def _():
                drain(ssem1, 1)

            accumulate(gbuf1, zbuf1)
            pltpu.async_copy(zbuf1, z_hbm.at[start + g1], ssem1)

            @pl.when(it + 1 < qh)
            def _():
                issue(g1 + 2, gbuf1, gsem1)

            return carry

        lax.fori_loop(0, qh, body, 0)
        drain(ssem0, 1)
        drain(ssem1, 1)

    do_type(idx_hbm0, z_hbm0, halves)


def _sc_gather_one(table, idx, nc, halves):
    mesh = plsc.VectorSubcoreMesh(core_axis_name="c", subcore_axis_name="s")
    out_type = jax.ShapeDtypeStruct((nc, C, 2 * H), jnp.float32)
    k = len(halves)
    nidx_max = ((nc * SC0_W) // (16 * (SC0_W + SC1_W))) * C * k
    scratch = [
        pltpu.VMEM((nidx_max,), jnp.int32),
        pltpu.VMEM((k * C, 128), jnp.float32),
        pltpu.VMEM((k * C, 128), jnp.float32),
        pltpu.VMEM((C, 2 * H), jnp.float32),
        pltpu.VMEM((C, 2 * H), jnp.float32),
        pltpu.SemaphoreType.DMA,
        pltpu.SemaphoreType.DMA,
        pltpu.SemaphoreType.DMA,
        pltpu.SemaphoreType.DMA,
    ]
    fn = pl.kernel(functools.partial(_sc_gather_body, halves),
                   out_type=out_type, mesh=mesh,
                   scratch_types=scratch,
                   compiler_params=pltpu.CompilerParams(
                       use_tc_tiling_on_sc=True))
    return fn(table, idx)


def _tail_body(z_ref, r_ref, wr_ref, b1_ref, w2_ref, b2_ref, w3_ref, b3_ref,
               o_ref):
    z = z_ref[...].reshape(M * C, 2 * H)
    zf = z[:, 0:H]
    zr = z[:, H:2 * H]
    base = r_ref[...] * wr_ref[...] + b1_ref[...]
    h1f = jnp.maximum(zf + base, 0.0)
    h1r = jnp.maximum(zr + base, 0.0)
    w2 = w2_ref[...]
    h2f = jnp.maximum(
        jnp.dot(h1f, w2, preferred_element_type=jnp.float32) + b2_ref[...], 0.0)
    h2r = jnp.maximum(
        jnp.dot(h1r, w2, preferred_element_type=jnp.float32) + b2_ref[...], 0.0)
    o_ref[...] = (jnp.dot(h2f + h2r, w3_ref[...],
                          preferred_element_type=jnp.float32) + b3_ref[...])


def _tail(z4, repr_, wr, b1, w2, b2, w3, b3):
    n = repr_.shape[0]
    grid = (n + M * C - 1) // (M * C)
    n_out = w3.shape[1]
    return pl.pallas_call(
        _tail_body,
        grid=(grid,),
        in_specs=[
            pl.BlockSpec((M, C, 2 * H), lambda i: (i, 0, 0)),
            pl.BlockSpec((M * C, 1), lambda i: (i, 0)),
            pl.BlockSpec((1, H), lambda i: (0, 0)),
            pl.BlockSpec((1, H), lambda i: (0, 0)),
            pl.BlockSpec((H, H), lambda i: (0, 0)),
            pl.BlockSpec((1, H), lambda i: (0, 0)),
            pl.BlockSpec((H, n_out), lambda i: (0, 0)),
            pl.BlockSpec((1, n_out), lambda i: (0, 0)),
        ],
        out_specs=pl.BlockSpec((M * C, n_out), lambda i: (i, 0)),
        out_shape=jax.ShapeDtypeStruct((n, n_out), jnp.float32),
    )(z4, repr_, wr, b1, w2, b2, w3, b3)


def _prep_idx(idx, offsets, npad, n_atoms):
    n, k = idx.shape
    off = jnp.asarray(offsets, jnp.int32)[None, :]
    shifted = idx.astype(jnp.int32) + off
    padv = (jnp.arange(npad - n, dtype=jnp.int32)[:, None] % n_atoms) + off
    return jnp.concatenate([shifted, padv], axis=0).T.reshape(-1)


def kernel(x, bond_idx, angle_idx, torsion_idx, bond_repr, angle_repr,
           torsion_repr, bond_params, angle_params, torsion_params):
    wb1 = bond_params[0]
    wa1 = angle_params[0]
    wt1 = torsion_params[0]
    na_ = x.shape[0]

    wstack = jnp.stack([
        jnp.concatenate([wb1[0:D], wb1[D:2 * D]], axis=1),
        jnp.concatenate([wa1[0:D], wa1[2 * D:3 * D]], axis=1),
        jnp.concatenate([wa1[D:2 * D], wa1[D:2 * D]], axis=1),
        jnp.concatenate([wt1[0:D], wt1[3 * D:4 * D]], axis=1),
        jnp.concatenate([wt1[D:2 * D], wt1[2 * D:3 * D]], axis=1),
    ])

    table = _project(x, wstack)

    nb, naf, nt = bond_idx.shape[0], angle_idx.shape[0], torsion_idx.shape[0]
    align = 32 * (SC0_W + SC1_W)
    step = C * align
    ncb = ((nb + step - 1) // step) * align
    nca = ((naf + step - 1) // step) * align
    nct = ((nt + step - 1) // step) * align

    def rows_padded(nc):
        return (nc + (nc * SC0_W) // (16 * (SC0_W + SC1_W))) * C

    bidx = _prep_idx(bond_idx, [0, 0], rows_padded(ncb), na_)
    aidx = _prep_idx(angle_idx, [na_, 2 * na_, na_], rows_padded(nca), na_)
    tidx = _prep_idx(torsion_idx, [3 * na_, 4 * na_, 4 * na_, 3 * na_],
                     rows_padded(nct), na_)

    zb = _sc_gather_one(table, bidx, ncb, HALVES["bond"])
    za = _sc_gather_one(table, aidx, nca, HALVES["angle"])
    zt = _sc_gather_one(table, tidx, nct, HALVES["torsion"])

    def tail_for(z4, repr_, params):
        w1, b1, w2, b2, w3, b3 = params
        wr = w1[-1:, :]
        return _tail(z4, repr_, wr, b1.reshape(1, H), w2, b2.reshape(1, H),
                     w3, (2.0 * b3).reshape(1, -1))

    ob = tail_for(zb, bond_repr, bond_params)
    oa = tail_for(za, angle_repr, angle_params)
    ot = tail_for(zt, torsion_repr, torsion_params)

    return jnp.concatenate([ob, oa, ot], axis=0)
